# baseline (speedup 1.0000x reference)
import functools

import jax
import jax.numpy as jnp
from jax import lax
from jax.experimental import pallas as pl
from jax.experimental.pallas import tpu as pltpu
from jax.experimental.pallas import tpu_sc as plsc

_EPS = 1e-12
_NC = 2
_NS = 16
_NW = _NC * _NS
_CH = 128
_NB = 5


def _tc_transpose_table(table_t):
    h, v = table_t.shape
    vc = 8192

    def body(x_ref, o_ref):
        eye = jnp.eye(h, dtype=jnp.float32)
        o_ref[...] = lax.dot_general(
            x_ref[...], eye, (((0,), (0,)), ((), ())),
            preferred_element_type=jnp.float32,
        )

    return pl.pallas_call(
        body,
        grid=(pl.cdiv(v, vc),),
        in_specs=[pl.BlockSpec((h, vc), lambda i: (0, i))],
        out_specs=pl.BlockSpec((vc, h), lambda i: (i, 0)),
        out_shape=jax.ShapeDtypeStruct((v, h), jnp.float32),
        compiler_params=pltpu.CompilerParams(vmem_limit_bytes=100 * 1024 * 1024),
    )(table_t)


def _sc_gather(table, idx3):
    nw, n_ch, ch = idx3.shape
    _, h = table.shape
    n = nw * n_ch * ch
    assert n_ch % _NB == 0 and n_ch // _NB >= 2
    mesh = plsc.VectorSubcoreMesh(core_axis_name="c", subcore_axis_name="s")

    @functools.partial(
        pl.kernel,
        mesh=mesh,
        compiler_params=pltpu.CompilerParams(use_tc_tiling_on_sc=False),
        out_type=jax.ShapeDtypeStruct((n, h), jnp.float32),
        scratch_types=[
            pltpu.VMEM((n_ch, ch), jnp.int32),
            pltpu.VMEM((_NB, ch, h), jnp.float32),
            pltpu.SemaphoreType.DMA((_NB,)),
        ],
    )
    def k(table_hbm, idx_hbm, out_hbm, idx_v, rows_v, gsem):
        c = lax.axis_index("c")
        s = lax.axis_index("s")
        wid = s * _NC + c
        base = wid * (n_ch * ch)
        pltpu.sync_copy(idx_hbm.at[wid], idx_v)

        for b in range(_NB):
            pltpu.async_copy(table_hbm.at[idx_v.at[b]], rows_v.at[b], gsem.at[b])

        def round_body(r, carry):
            j0 = r * _NB
            for b in range(_NB):
                pltpu.make_async_copy(
                    table_hbm.at[idx_v.at[b]], rows_v.at[b], gsem.at[b]
                ).wait()
                pltpu.sync_copy(rows_v.at[b], out_hbm.at[pl.ds(base + (j0 + b) * ch, ch)])
                pltpu.async_copy(
                    table_hbm.at[idx_v.at[j0 + b + _NB]], rows_v.at[b], gsem.at[b]
                )
            return carry

        n_rounds = n_ch // _NB - 1
        lax.fori_loop(0, n_rounds, round_body, 0)

        j0 = n_rounds * _NB
        for b in range(_NB):
            pltpu.make_async_copy(
                table_hbm.at[idx_v.at[b]], rows_v.at[b], gsem.at[b]
            ).wait()
            pltpu.sync_copy(rows_v.at[b], out_hbm.at[pl.ds(base + (j0 + b) * ch, ch)])

    return k(table, idx3)


def _tc_ln(x, pos, tte, lnw, lnb):
    b, s, h = x.shape
    sb = 8

    def body(x_ref, pos_ref, tte_ref, w_ref, b_ref, o_ref):
        bias = pos_ref[...] + tte_ref[0:1, :]
        xx = x_ref[...] + bias[None]
        mu = jnp.mean(xx, axis=-1, keepdims=True)
        xc = xx - mu
        var = jnp.mean(xc * xc, axis=-1, keepdims=True)
        y = xc * lax.rsqrt(var + _EPS) * w_ref[...] + b_ref[...]
        eye = jnp.eye(y.shape[-1], dtype=jnp.float32)
        for j in range(y.shape[1]):
            o_ref[j] = lax.dot_general(
                eye, y[:, j, :], (((1,), (1,)), ((), ())),
                preferred_element_type=jnp.float32,
            )

    return pl.pallas_call(
        body,
        grid=(s // sb,),
        in_specs=[
            pl.BlockSpec((b, sb, h), lambda i: (0, i, 0)),
            pl.BlockSpec((sb, h), lambda i: (i, 0)),
            pl.BlockSpec(tte.shape, lambda i: (0, 0)),
            pl.BlockSpec((1, h), lambda i: (0, 0)),
            pl.BlockSpec((1, h), lambda i: (0, 0)),
        ],
        out_specs=pl.BlockSpec((sb, h, b), lambda i: (i, 0, 0)),
        out_shape=jax.ShapeDtypeStruct((s, h, b), jnp.float32),
        compiler_params=pltpu.CompilerParams(vmem_limit_bytes=100 * 1024 * 1024),
    )(x, pos, tte, lnw, lnb)


def kernel(input_ids, word_embeddings, position_embeddings, token_type_embeddings, ln_weight, ln_bias):
    b, s = input_ids.shape
    v, h = word_embeddings.shape
    n = b * s
    per_w = n // _NW
    n_ch = per_w // _CH
    assert per_w * _NW == n and n_ch * _CH == per_w
    idx3 = input_ids.astype(jnp.int32).reshape(_NW, n_ch, _CH)
    table_rm = _tc_transpose_table(jnp.transpose(word_embeddings))
    g = _sc_gather(table_rm, idx3)
    out_shb = _tc_ln(
        g.reshape(b, s, h),
        position_embeddings[:s],
        token_type_embeddings,
        ln_weight.reshape(1, h),
        ln_bias.reshape(1, h),
    )
    return jnp.transpose(out_shb, (2, 0, 1))


# speedup vs baseline: 1.7710x; 1.7710x over previous
import functools

import jax
import jax.numpy as jnp
from jax import lax
from jax.experimental import pallas as pl
from jax.experimental.pallas import tpu as pltpu
from jax.experimental.pallas import tpu_sc as plsc

_EPS = 1e-12
_NC = 2
_NS = 16
_NW = _NC * _NS
_CH = 128
_NB = 5
_D = 512000
_VC2 = 4096


def _tc_build_paired(table_t):
    h, v = table_t.shape
    n_blk = _D // _VC2
    last_blk = pl.cdiv(v, _VC2) - 1

    def body(x1_ref, x2_ref, o_ref):
        eye = jnp.eye(h, dtype=jnp.float32)
        o_ref[:, 0:h] = lax.dot_general(
            x1_ref[...], eye, (((0,), (0,)), ((), ())),
            preferred_element_type=jnp.float32,
        )
        o_ref[:, h:2 * h] = lax.dot_general(
            x2_ref[...], eye, (((0,), (0,)), ((), ())),
            preferred_element_type=jnp.float32,
        )

    return pl.pallas_call(
        body,
        grid=(n_blk,),
        in_specs=[
            pl.BlockSpec((h, _VC2), lambda i: (0, i)),
            pl.BlockSpec((h, _VC2), lambda i: (0, jnp.minimum(i + n_blk, last_blk))),
        ],
        out_specs=pl.BlockSpec((_VC2, 2 * h), lambda i: (i, 0)),
        out_shape=jax.ShapeDtypeStruct((_D, 2 * h), jnp.float32),
        compiler_params=pltpu.CompilerParams(vmem_limit_bytes=100 * 1024 * 1024),
    )(table_t, table_t)


def _sc_gather(table, idx3):
    nw, n_ch, ch = idx3.shape
    _, h = table.shape
    n = nw * n_ch * ch
    assert n_ch % _NB == 0 and n_ch // _NB >= 2
    mesh = plsc.VectorSubcoreMesh(core_axis_name="c", subcore_axis_name="s")

    @functools.partial(
        pl.kernel,
        mesh=mesh,
        compiler_params=pltpu.CompilerParams(use_tc_tiling_on_sc=False),
        out_type=jax.ShapeDtypeStruct((n, h), jnp.float32),
        scratch_types=[
            pltpu.VMEM((n_ch, ch), jnp.int32),
            pltpu.VMEM((n_ch, ch), jnp.int32),
            pltpu.VMEM((_NB, ch, h), jnp.float32),
            pltpu.SemaphoreType.DMA((_NB,)),
        ],
    )
    def k(table_hbm, idx_hbm, out_hbm, idx_v, pidx_v, rows_v, gsem):
        c = lax.axis_index("c")
        s = lax.axis_index("s")
        wid = s * _NC + c
        base = wid * (n_ch * ch)
        pltpu.sync_copy(idx_hbm.at[wid], idx_v)

        def to_paired(j):
            for kk in range(ch // 16):
                a = idx_v[j, pl.ds(kk * 16, 16)]
                q = jnp.where(a < _D, a + a, a + a - (2 * _D - 1))
                pidx_v[j, pl.ds(kk * 16, 16)] = q

        for b in range(_NB):
            to_paired(b)
            pltpu.async_copy(table_hbm.at[pidx_v.at[b]], rows_v.at[b], gsem.at[b])

        def round_body(r, carry):
            j0 = r * _NB
            for b in range(_NB):
                pltpu.make_async_copy(
                    table_hbm.at[pidx_v.at[b]], rows_v.at[b], gsem.at[b]
                ).wait()
                pltpu.sync_copy(rows_v.at[b], out_hbm.at[pl.ds(base + (j0 + b) * ch, ch)])
                to_paired(j0 + b + _NB)
                pltpu.async_copy(
                    table_hbm.at[pidx_v.at[j0 + b + _NB]], rows_v.at[b], gsem.at[b]
                )
            return carry

        n_rounds = n_ch // _NB - 1
        lax.fori_loop(0, n_rounds, round_body, 0)

        j0 = n_rounds * _NB
        for b in range(_NB):
            pltpu.make_async_copy(
                table_hbm.at[pidx_v.at[b]], rows_v.at[b], gsem.at[b]
            ).wait()
            pltpu.sync_copy(rows_v.at[b], out_hbm.at[pl.ds(base + (j0 + b) * ch, ch)])

    return k(table, idx3)


def _tc_ln(x, pos, tte, lnw, lnb):
    b, s, h = x.shape
    sb = 8

    def body(x_ref, pos_ref, tte_ref, w_ref, b_ref, o_ref):
        bias = pos_ref[...] + tte_ref[0:1, :]
        xx = x_ref[...] + bias[None]
        mu = jnp.mean(xx, axis=-1, keepdims=True)
        xc = xx - mu
        var = jnp.mean(xc * xc, axis=-1, keepdims=True)
        y = xc * lax.rsqrt(var + _EPS) * w_ref[...] + b_ref[...]
        eye = jnp.eye(y.shape[-1], dtype=jnp.float32)
        for j in range(y.shape[1]):
            o_ref[j] = lax.dot_general(
                eye, y[:, j, :], (((1,), (1,)), ((), ())),
                preferred_element_type=jnp.float32,
            )

    return pl.pallas_call(
        body,
        grid=(s // sb,),
        in_specs=[
            pl.BlockSpec((b, sb, h), lambda i: (0, i, 0)),
            pl.BlockSpec((sb, h), lambda i: (i, 0)),
            pl.BlockSpec(tte.shape, lambda i: (0, 0)),
            pl.BlockSpec((1, h), lambda i: (0, 0)),
            pl.BlockSpec((1, h), lambda i: (0, 0)),
        ],
        out_specs=pl.BlockSpec((sb, h, b), lambda i: (i, 0, 0)),
        out_shape=jax.ShapeDtypeStruct((s, h, b), jnp.float32),
        compiler_params=pltpu.CompilerParams(vmem_limit_bytes=100 * 1024 * 1024),
    )(x, pos, tte, lnw, lnb)


def kernel(input_ids, word_embeddings, position_embeddings, token_type_embeddings, ln_weight, ln_bias):
    b, s = input_ids.shape
    v, h = word_embeddings.shape
    n = b * s
    per_w = n // _NW
    n_ch = per_w // _CH
    assert per_w * _NW == n and n_ch * _CH == per_w
    idx3 = input_ids.astype(jnp.int32).reshape(_NW, n_ch, _CH)
    paired = _tc_build_paired(jnp.transpose(word_embeddings))
    g = _sc_gather(paired.reshape(2 * _D, h), idx3)
    out_shb = _tc_ln(
        g.reshape(b, s, h),
        position_embeddings[:s],
        token_type_embeddings,
        ln_weight.reshape(1, h),
        ln_bias.reshape(1, h),
    )
    return jnp.transpose(out_shb, (2, 0, 1))
